# Initial kernel scaffold; baseline (speedup 1.0000x reference)
import jax
import jax.numpy as jnp
from jax.experimental import pallas as pl


def kernel(z, edge_index, edge_type, W_lin, b_lin, w1, q1, k1, b1, w2, q2, k2, b2):
    raise NotImplementedError("write your pallas kernel here")


# speedup vs baseline: 29.2905x; 29.2905x over previous
import functools

import jax
import jax.numpy as jnp
from jax import lax
from jax.experimental import pallas as pl
from jax.experimental.pallas import tpu as pltpu
from jax.experimental.pallas import tpu_sc as plsc

N = 10000
E = 320000
R = 4
D = 128

NC = 2
NS = 16
NW = NC * NS
EPT = E // NW
SUP = 2000
CH = 80
NCHUNK = SUP // CH
NSUP = EPT // SUP
NRC = N // CH

_GDN = lax.GatherDimensionNumbers(
    offset_dims=(), collapsed_slice_dims=(0,), start_index_map=(0,))


def _linear_body(z_ref, w_ref, b_ref, o_ref):
    o_ref[...] = lax.dot_general(
        z_ref[...], w_ref[...], (((1,), (1,)), ((), ())),
        preferred_element_type=jnp.float32) + b_ref[...][None, :]


def _linear(z, w, b):
    return pl.pallas_call(
        _linear_body,
        out_shape=jax.ShapeDtypeStruct((N, D), jnp.float32),
    )(z, w, b)


def _prep_body(h_ref, w_ref, q_ref, k_ref, xt_ref, sq_ref, sk_ref, sh_ref):
    r = pl.program_id(0)
    xt = jnp.dot(h_ref[...], w_ref[0], preferred_element_type=jnp.float32)
    xt_ref[0] = xt
    sq = jnp.dot(xt, q_ref[...][:, 0], preferred_element_type=jnp.float32)
    sk = jnp.dot(xt, k_ref[...][:, 0], preferred_element_type=jnp.float32)
    sq_ref[0, 0] = sq
    sk_ref[0, 0] = sk
    cur = jnp.max(sq) + jnp.max(sk)

    @pl.when(r == 0)
    def _():
        sh_ref[...] = jnp.full((8, 128), -jnp.inf, jnp.float32)

    sh_ref[...] = jnp.maximum(sh_ref[...], cur)

    @pl.when(r == R - 1)
    def _():
        v = sh_ref[...]
        sh_ref[...] = jnp.where(v >= 0, v, 0.2 * v)


def _prep(h, w, q, k):
    return pl.pallas_call(
        _prep_body,
        grid=(R,),
        in_specs=[
            pl.BlockSpec((N, D), lambda r: (0, 0)),
            pl.BlockSpec((1, D, D), lambda r: (r, 0, 0)),
            pl.BlockSpec((D, 1), lambda r: (0, 0)),
            pl.BlockSpec((D, 1), lambda r: (0, 0)),
        ],
        out_specs=[
            pl.BlockSpec((1, N, D), lambda r: (r, 0, 0)),
            pl.BlockSpec((1, 1, N), lambda r: (r, 0, 0)),
            pl.BlockSpec((1, 1, N), lambda r: (r, 0, 0)),
            pl.BlockSpec((8, 128), lambda r: (0, 0)),
        ],
        out_shape=[
            jax.ShapeDtypeStruct((R, N, D), jnp.float32),
            jax.ShapeDtypeStruct((R, 1, N), jnp.float32),
            jax.ShapeDtypeStruct((R, 1, N), jnp.float32),
            jax.ShapeDtypeStruct((8, 128), jnp.float32),
        ],
    )(h, w, q, k)


def _finalize_body(acc_ref, den_ref, b_ref, o_ref, *, do_relu):
    s = acc_ref[0] + acc_ref[1]
    dd = den_ref[0] + den_ref[1]
    o = s / (dd[:, None] + 1e-16) + b_ref[...][None, :]
    if do_relu:
        o = jnp.maximum(o, 0.0)
    o_ref[...] = o


def _finalize(acc, den, b, do_relu):
    return pl.pallas_call(
        functools.partial(_finalize_body, do_relu=do_relu),
        out_shape=jax.ShapeDtypeStruct((N, D), jnp.float32),
    )(acc, den, b)


def _edge_body(esrc_ref, edst_ref, et_ref, sq_ref, sk_ref, sh_ref, xt_ref,
               acc_ref, den_ref,
               shv_v, src_v, dst_v, ty_v, rows_v, den_v,
               exb, siv, sjv, qidx, ridx, didx, acc_sh, den_sh, sem):
    c = lax.axis_index("c")
    s = lax.axis_index("s")
    g = c * NS + s
    zero16 = jnp.zeros((16,), jnp.float32)
    lane = lax.iota(jnp.int32, 16)

    pltpu.sync_copy(sh_ref.at[0, pl.ds(0, 16)], shv_v)

    def _zr(i, carry):
        for j in range(D // 16):
            rows_v[i, pl.ds(j * 16, 16)] = zero16
        return carry
    lax.fori_loop(0, CH, _zr, 0)
    for j in range(-(-NRC // NS)):
        cid = s + j * NS

        @pl.when(cid < NRC)
        def _():
            pltpu.sync_copy(rows_v, acc_sh.at[pl.ds(cid * CH, CH)])

    @pl.when(s == 0)
    def _():
        def _zd(i, carry):
            den_v[pl.ds(i * 16, 16)] = zero16
            return carry
        lax.fori_loop(0, N // 16, _zd, 0)
        pltpu.sync_copy(den_v, den_sh)

    plsc.subcore_barrier()

    shv = shv_v[...]

    def _sup(sup, carry0):
        b0 = g * EPT + sup * SUP
        pltpu.sync_copy(esrc_ref.at[pl.ds(b0, SUP)], src_v)
        pltpu.sync_copy(edst_ref.at[pl.ds(b0, SUP)], dst_v)
        pltpu.sync_copy(et_ref.at[pl.ds(b0, SUP)], ty_v)

        def _chunk(ch, carry):
            cb = ch * CH
            for gi in range(CH // 16):
                o = cb + gi * 16
                sv = src_v[pl.ds(o, 16)]
                dv = dst_v[pl.ds(o, 16)]
                tv = ty_v[pl.ds(o, 16)]
                tn = tv * N
                qidx[0, pl.ds(gi * 16, 16)] = tn + dv
                ridx[0, pl.ds(gi * 16, 16)] = tn + sv
                didx[0, pl.ds(gi * 16, 16)] = dv
            pltpu.async_copy(sq_ref.at[qidx.at[0]], siv, sem).wait()
            pltpu.async_copy(sk_ref.at[ridx.at[0]], sjv, sem).wait()
            pltpu.async_copy(xt_ref.at[ridx.at[0]], rows_v, sem).wait()

            for gi in range(CH // 16):
                si = siv[pl.ds(gi * 16, 16)]
                sj = sjv[pl.ds(gi * 16, 16)]
                al = si + sj
                al = jnp.where(al >= 0, al, 0.2 * al)
                exb[pl.ds(gi * 16, 16)] = jnp.exp(al - shv)

            def _scale(g2, carry2):
                ex16 = exb[pl.ds(g2 * 16, 16)]
                for j in range(16):
                    m16 = lax.gather(
                        ex16, jnp.full((16, 1), j, jnp.int32),
                        _GDN, (1,),
                        mode=lax.GatherScatterMode.PROMISE_IN_BOUNDS)
                    ci = g2 * 16 + j
                    for jj in range(D // 16):
                        rows_v[ci, pl.ds(jj * 16, 16)] = (
                            rows_v[ci, pl.ds(jj * 16, 16)] * m16)
                return carry2
            lax.fori_loop(0, CH // 16, _scale, 0)

            pltpu.sync_copy(rows_v, acc_sh.at[didx.at[0]], add=True)
            pltpu.sync_copy(exb, den_sh.at[didx.at[0]], add=True)
            return carry
        lax.fori_loop(0, NCHUNK, _chunk, 0)
        return carry0
    lax.fori_loop(0, NSUP, _sup, 0)

    plsc.subcore_barrier()

    for j in range(-(-NRC // NS)):
        cid = s + j * NS

        @pl.when(cid < NRC)
        def _():
            pltpu.sync_copy(acc_sh.at[pl.ds(cid * CH, CH)], rows_v)
            pltpu.sync_copy(rows_v, acc_ref.at[c, pl.ds(cid * CH, CH)])

    @pl.when(s == 0)
    def _():
        pltpu.sync_copy(den_sh, den_v)
        pltpu.sync_copy(den_v, den_ref.at[pl.ds(c * N, N)])


def _edge_pass(esrc, edst, et, sqf, skf, shift, xt2):
    mesh = plsc.VectorSubcoreMesh(core_axis_name="c", subcore_axis_name="s")
    f = pl.kernel(
        _edge_body,
        mesh=mesh,
        out_type=[
            jax.ShapeDtypeStruct((NC, N, D), jnp.float32),
            jax.ShapeDtypeStruct((NC * N,), jnp.float32),
        ],
        scratch_types=[
            pltpu.VMEM((16,), jnp.float32),
            pltpu.VMEM((SUP,), jnp.int32),
            pltpu.VMEM((SUP,), jnp.int32),
            pltpu.VMEM((SUP,), jnp.int32),
            pltpu.VMEM((CH, D), jnp.float32),
            pltpu.VMEM((N,), jnp.float32),
            pltpu.VMEM((CH,), jnp.float32),
            pltpu.VMEM((CH,), jnp.float32),
            pltpu.VMEM((CH,), jnp.float32),
            pltpu.VMEM((1, CH), jnp.int32),
            pltpu.VMEM((1, CH), jnp.int32),
            pltpu.VMEM((1, CH), jnp.int32),
            pltpu.VMEM_SHARED((N, D), jnp.float32),
            pltpu.VMEM_SHARED((N,), jnp.float32),
            pltpu.SemaphoreType.DMA,
        ],
    )
    return f(esrc, edst, et, sqf, skf, shift, xt2)


def _rgat_layer(h, ei, et, w, q, k, b, do_relu):
    xt, sq3, sk3, shift = _prep(h, w, q, k)
    xt2 = xt.reshape(R * N, D)
    acc, den = _edge_pass(ei[0], ei[1], et, sq3.reshape(R * N),
                          sk3.reshape(R * N), shift, xt2)
    return _finalize(acc, den.reshape(NC, N), b, do_relu)


def kernel(z, edge_index, edge_type, W_lin, b_lin, w1, q1, k1, b1,
           w2, q2, k2, b2):
    h = _linear(z, W_lin, b_lin)
    h = _rgat_layer(h, edge_index, edge_type, w1, q1, k1, b1, True)
    return _rgat_layer(h, edge_index, edge_type, w2, q2, k2, b2, False)


# speedup vs baseline: 61.9576x; 2.1153x over previous
import functools

import jax
import jax.numpy as jnp
from jax import lax
from jax.experimental import pallas as pl
from jax.experimental.pallas import tpu as pltpu
from jax.experimental.pallas import tpu_sc as plsc

N = 10000
E = 320000
R = 4
D = 128

NC = 2
NS = 16
NW = NC * NS
EPT = E // NW
SUP = 2000
CH = 80
NCHUNK = SUP // CH
NSUP = EPT // SUP
NRC = N // CH

_GDN = lax.GatherDimensionNumbers(
    offset_dims=(), collapsed_slice_dims=(0,), start_index_map=(0,))


def _linear_body(z_ref, w_ref, b_ref, o_ref):
    o_ref[...] = lax.dot_general(
        z_ref[...], w_ref[...], (((1,), (1,)), ((), ())),
        preferred_element_type=jnp.float32) + b_ref[...][None, :]


def _linear(z, w, b):
    return pl.pallas_call(
        _linear_body,
        out_shape=jax.ShapeDtypeStruct((N, D), jnp.float32),
    )(z, w, b)


def _prep_body(h_ref, w_ref, q_ref, k_ref, xt_ref, sq_ref, sk_ref, sh_ref):
    r = pl.program_id(0)
    xt = jnp.dot(h_ref[...], w_ref[0], preferred_element_type=jnp.float32)
    xt_ref[0] = xt
    sq = jnp.dot(xt, q_ref[...][:, 0], preferred_element_type=jnp.float32)
    sk = jnp.dot(xt, k_ref[...][:, 0], preferred_element_type=jnp.float32)
    sq_ref[0, 0] = sq
    sk_ref[0, 0] = sk
    cur = jnp.max(sq) + jnp.max(sk)

    @pl.when(r == 0)
    def _():
        sh_ref[...] = jnp.full((8, 128), -jnp.inf, jnp.float32)

    sh_ref[...] = jnp.maximum(sh_ref[...], cur)

    @pl.when(r == R - 1)
    def _():
        v = sh_ref[...]
        sh_ref[...] = jnp.where(v >= 0, v, 0.2 * v)


def _prep(h, w, q, k):
    return pl.pallas_call(
        _prep_body,
        grid=(R,),
        in_specs=[
            pl.BlockSpec((N, D), lambda r: (0, 0)),
            pl.BlockSpec((1, D, D), lambda r: (r, 0, 0)),
            pl.BlockSpec((D, 1), lambda r: (0, 0)),
            pl.BlockSpec((D, 1), lambda r: (0, 0)),
        ],
        out_specs=[
            pl.BlockSpec((1, N, D), lambda r: (r, 0, 0)),
            pl.BlockSpec((1, 1, N), lambda r: (r, 0, 0)),
            pl.BlockSpec((1, 1, N), lambda r: (r, 0, 0)),
            pl.BlockSpec((8, 128), lambda r: (0, 0)),
        ],
        out_shape=[
            jax.ShapeDtypeStruct((R, N, D), jnp.float32),
            jax.ShapeDtypeStruct((R, 1, N), jnp.float32),
            jax.ShapeDtypeStruct((R, 1, N), jnp.float32),
            jax.ShapeDtypeStruct((8, 128), jnp.float32),
        ],
    )(h, w, q, k)


def _finalize_body(acc_ref, den_ref, b_ref, o_ref, *, do_relu):
    s = acc_ref[0] + acc_ref[1]
    dd = den_ref[0] + den_ref[1]
    o = s / (dd[:, None] + 1e-16) + b_ref[...][None, :]
    if do_relu:
        o = jnp.maximum(o, 0.0)
    o_ref[...] = o


def _finalize(acc, den, b, do_relu):
    return pl.pallas_call(
        functools.partial(_finalize_body, do_relu=do_relu),
        out_shape=jax.ShapeDtypeStruct((N, D), jnp.float32),
    )(acc, den, b)


def _edge_body(esrc_ref, edst_ref, et_ref, sq_ref, sk_ref, sh_ref, xt_ref,
               acc_ref, den_ref,
               shv_v, src_v, dst_v, ty_v, rows0, rows1, den_v,
               exb0, exb1, siv0, siv1, sjv0, sjv1,
               qidx0, qidx1, ridx0, ridx1, didx0, didx1,
               acc_sh, den_sh, sem_g0, sem_g1, sem_s0, sem_s1):
    c = lax.axis_index("c")
    s = lax.axis_index("s")
    g = c * NS + s
    zero16 = jnp.zeros((16,), jnp.float32)
    bufs = ((rows0, exb0, siv0, sjv0, qidx0, ridx0, didx0, sem_g0, sem_s0),
            (rows1, exb1, siv1, sjv1, qidx1, ridx1, didx1, sem_g1, sem_s1))

    pltpu.sync_copy(sh_ref.at[0, pl.ds(0, 16)], shv_v)

    def _zr(i, carry):
        for j in range(D // 16):
            rows0[i, pl.ds(j * 16, 16)] = zero16
        return carry
    lax.fori_loop(0, CH, _zr, 0)
    for j in range(-(-NRC // NS)):
        cid = s + j * NS

        @pl.when(cid < NRC)
        def _():
            pltpu.sync_copy(rows0, acc_sh.at[pl.ds(cid * CH, CH)])

    @pl.when(s == 0)
    def _():
        def _zd(i, carry):
            den_v[pl.ds(i * 16, 16)] = zero16
            return carry
        lax.fori_loop(0, N // 16, _zd, 0)
        pltpu.sync_copy(den_v, den_sh)

    plsc.subcore_barrier()

    shv = shv_v[...]

    def _wait_scatters(b):
        rows_v, exb, _, _, _, _, didx, _, sem_s = bufs[b]
        pltpu.make_async_copy(rows_v, acc_sh.at[didx.at[0]], sem_s).wait()
        pltpu.make_async_copy(exb, den_sh.at[didx.at[0]], sem_s).wait()

    def _stage_a(i, b, guard):
        rows_v, exb, siv, sjv, qidx, ridx, didx, sem_g, _ = bufs[b]
        if guard:
            @pl.when(i >= 2)
            def _():
                _wait_scatters(b)
        cb = i * CH
        for gi in range(CH // 16):
            o = cb + gi * 16
            sv = src_v[pl.ds(o, 16)]
            dv = dst_v[pl.ds(o, 16)]
            tv = ty_v[pl.ds(o, 16)]
            tn = tv * N
            qidx[0, pl.ds(gi * 16, 16)] = tn + dv
            ridx[0, pl.ds(gi * 16, 16)] = tn + sv
            didx[0, pl.ds(gi * 16, 16)] = dv
        pltpu.async_copy(sq_ref.at[qidx.at[0]], siv, sem_g)
        pltpu.async_copy(sk_ref.at[ridx.at[0]], sjv, sem_g)
        pltpu.async_copy(xt_ref.at[ridx.at[0]], rows_v, sem_g)

    def _stage_b(i, b):
        rows_v, exb, siv, sjv, qidx, ridx, didx, sem_g, sem_s = bufs[b]
        pltpu.make_async_copy(sq_ref.at[qidx.at[0]], siv, sem_g).wait()
        pltpu.make_async_copy(sk_ref.at[ridx.at[0]], sjv, sem_g).wait()
        pltpu.make_async_copy(xt_ref.at[ridx.at[0]], rows_v, sem_g).wait()

        for gi in range(CH // 16):
            si = siv[pl.ds(gi * 16, 16)]
            sj = sjv[pl.ds(gi * 16, 16)]
            al = si + sj
            al = jnp.where(al >= 0, al, 0.2 * al)
            exb[pl.ds(gi * 16, 16)] = jnp.exp(al - shv)

        def _scale(g2, carry2):
            ex16 = exb[pl.ds(g2 * 16, 16)]
            for j in range(16):
                m16 = lax.gather(
                    ex16, jnp.full((16, 1), j, jnp.int32),
                    _GDN, (1,),
                    mode=lax.GatherScatterMode.PROMISE_IN_BOUNDS)
                ci = g2 * 16 + j
                for jj in range(D // 16):
                    rows_v[ci, pl.ds(jj * 16, 16)] = (
                        rows_v[ci, pl.ds(jj * 16, 16)] * m16)
            return carry2
        lax.fori_loop(0, CH // 16, _scale, 0)

        pltpu.async_copy(rows_v, acc_sh.at[didx.at[0]], sem_s, add=True)
        pltpu.async_copy(exb, den_sh.at[didx.at[0]], sem_s, add=True)

    def _sup(sup, carry0):
        b0 = g * EPT + sup * SUP
        pltpu.sync_copy(esrc_ref.at[pl.ds(b0, SUP)], src_v)
        pltpu.sync_copy(edst_ref.at[pl.ds(b0, SUP)], dst_v)
        pltpu.sync_copy(et_ref.at[pl.ds(b0, SUP)], ty_v)

        _stage_a(0, 0, False)

        def _pair(p, c2):
            _stage_a(2 * p + 1, 1, True)
            _stage_b(2 * p, 0)
            _stage_a(2 * p + 2, 0, True)
            _stage_b(2 * p + 1, 1)
            return c2
        lax.fori_loop(0, (NCHUNK - 1) // 2, _pair, 0)
        _stage_b(NCHUNK - 1, 0)
        _wait_scatters(1)
        _wait_scatters(0)
        return carry0
    lax.fori_loop(0, NSUP, _sup, 0)

    plsc.subcore_barrier()

    for j in range(-(-NRC // NS)):
        cid = s + j * NS

        @pl.when(cid < NRC)
        def _():
            pltpu.sync_copy(acc_sh.at[pl.ds(cid * CH, CH)], rows0)
            pltpu.sync_copy(rows0, acc_ref.at[c, pl.ds(cid * CH, CH)])

    @pl.when(s == 0)
    def _():
        pltpu.sync_copy(den_sh, den_v)
        pltpu.sync_copy(den_v, den_ref.at[pl.ds(c * N, N)])


def _edge_pass(esrc, edst, et, sqf, skf, shift, xt2):
    mesh = plsc.VectorSubcoreMesh(core_axis_name="c", subcore_axis_name="s")
    f = pl.kernel(
        _edge_body,
        mesh=mesh,
        out_type=[
            jax.ShapeDtypeStruct((NC, N, D), jnp.float32),
            jax.ShapeDtypeStruct((NC * N,), jnp.float32),
        ],
        scratch_types=[
            pltpu.VMEM((16,), jnp.float32),
            pltpu.VMEM((SUP,), jnp.int32),
            pltpu.VMEM((SUP,), jnp.int32),
            pltpu.VMEM((SUP,), jnp.int32),
            pltpu.VMEM((CH, D), jnp.float32),
            pltpu.VMEM((CH, D), jnp.float32),
            pltpu.VMEM((N,), jnp.float32),
            pltpu.VMEM((CH,), jnp.float32),
            pltpu.VMEM((CH,), jnp.float32),
            pltpu.VMEM((CH,), jnp.float32),
            pltpu.VMEM((CH,), jnp.float32),
            pltpu.VMEM((CH,), jnp.float32),
            pltpu.VMEM((CH,), jnp.float32),
            pltpu.VMEM((1, CH), jnp.int32),
            pltpu.VMEM((1, CH), jnp.int32),
            pltpu.VMEM((1, CH), jnp.int32),
            pltpu.VMEM((1, CH), jnp.int32),
            pltpu.VMEM((1, CH), jnp.int32),
            pltpu.VMEM((1, CH), jnp.int32),
            pltpu.VMEM_SHARED((N, D), jnp.float32),
            pltpu.VMEM_SHARED((N,), jnp.float32),
            pltpu.SemaphoreType.DMA,
            pltpu.SemaphoreType.DMA,
            pltpu.SemaphoreType.DMA,
            pltpu.SemaphoreType.DMA,
        ],
    )
    return f(esrc, edst, et, sqf, skf, shift, xt2)


def _rgat_layer(h, ei, et, w, q, k, b, do_relu):
    xt, sq3, sk3, shift = _prep(h, w, q, k)
    xt2 = xt.reshape(R * N, D)
    acc, den = _edge_pass(ei[0], ei[1], et, sq3.reshape(R * N),
                          sk3.reshape(R * N), shift, xt2)
    return _finalize(acc, den.reshape(NC, N), b, do_relu)


def kernel(z, edge_index, edge_type, W_lin, b_lin, w1, q1, k1, b1,
           w2, q2, k2, b2):
    h = _linear(z, W_lin, b_lin)
    h = _rgat_layer(h, edge_index, edge_type, w1, q1, k1, b1, True)
    return _rgat_layer(h, edge_index, edge_type, w2, q2, k2, b2, False)


# speedup vs baseline: 64.8263x; 1.0463x over previous
import functools

import jax
import jax.numpy as jnp
from jax import lax
from jax.experimental import pallas as pl
from jax.experimental.pallas import tpu as pltpu
from jax.experimental.pallas import tpu_sc as plsc

N = 10000
E = 320000
R = 4
D = 128

NC = 2
NS = 16
NW = NC * NS
EPT = E // NW
SUP = 2000
CH = 80
NCHUNK = SUP // CH
NSUP = EPT // SUP
NRC = N // CH

_GDN = lax.GatherDimensionNumbers(
    offset_dims=(), collapsed_slice_dims=(0,), start_index_map=(0,))


def _linear_body(z_ref, w_ref, b_ref, o_ref):
    o_ref[...] = lax.dot_general(
        z_ref[...], w_ref[...], (((1,), (1,)), ((), ())),
        preferred_element_type=jnp.float32) + b_ref[...][None, :]


def _linear(z, w, b):
    return pl.pallas_call(
        _linear_body,
        out_shape=jax.ShapeDtypeStruct((N, D), jnp.float32),
    )(z, w, b)


def _prep_body(h_ref, w_ref, q_ref, k_ref, xt_ref, sq_ref, sk_ref, sh_ref):
    r = pl.program_id(0)
    xt = jnp.dot(h_ref[...], w_ref[0], preferred_element_type=jnp.float32)
    xt_ref[0] = xt
    sq = jnp.dot(xt, q_ref[...][:, 0], preferred_element_type=jnp.float32)
    sk = jnp.dot(xt, k_ref[...][:, 0], preferred_element_type=jnp.float32)
    sq_ref[0, 0] = sq
    sk_ref[0, 0] = sk
    cur = jnp.max(sq) + jnp.max(sk)

    @pl.when(r == 0)
    def _():
        sh_ref[...] = jnp.full((8, 128), -jnp.inf, jnp.float32)

    sh_ref[...] = jnp.maximum(sh_ref[...], cur)

    @pl.when(r == R - 1)
    def _():
        v = sh_ref[...]
        sh_ref[...] = jnp.where(v >= 0, v, 0.2 * v)


def _prep(h, w, q, k):
    return pl.pallas_call(
        _prep_body,
        grid=(R,),
        in_specs=[
            pl.BlockSpec((N, D), lambda r: (0, 0)),
            pl.BlockSpec((1, D, D), lambda r: (r, 0, 0)),
            pl.BlockSpec((D, 1), lambda r: (0, 0)),
            pl.BlockSpec((D, 1), lambda r: (0, 0)),
        ],
        out_specs=[
            pl.BlockSpec((1, N, D), lambda r: (r, 0, 0)),
            pl.BlockSpec((1, 1, N), lambda r: (r, 0, 0)),
            pl.BlockSpec((1, 1, N), lambda r: (r, 0, 0)),
            pl.BlockSpec((8, 128), lambda r: (0, 0)),
        ],
        out_shape=[
            jax.ShapeDtypeStruct((R, N, D), jnp.float32),
            jax.ShapeDtypeStruct((R, 1, N), jnp.float32),
            jax.ShapeDtypeStruct((R, 1, N), jnp.float32),
            jax.ShapeDtypeStruct((8, 128), jnp.float32),
        ],
    )(h, w, q, k)


def _finalize_body(acc_ref, den_ref, b_ref, o_ref, *, do_relu):
    s = acc_ref[0] + acc_ref[1]
    dd = den_ref[0] + den_ref[1]
    o = s / (dd[:, None] + 1e-16) + b_ref[...][None, :]
    if do_relu:
        o = jnp.maximum(o, 0.0)
    o_ref[...] = o


def _finalize(acc, den, b, do_relu):
    return pl.pallas_call(
        functools.partial(_finalize_body, do_relu=do_relu),
        out_shape=jax.ShapeDtypeStruct((N, D), jnp.float32),
    )(acc, den, b)


def _edge_body(esrc_ref, edst_ref, et_ref, sq_ref, sk_ref, sh_ref, xt_ref,
               acc_ref, den_ref,
               shv_v, src_v, dst_v, ty_v, rows0, rows1, den_v,
               exb0, exb1, siv0, siv1, sjv0, sjv1,
               qidx0, qidx1, ridx0, ridx1, didx0, didx1,
               acc_sh, den_sh, sqs_sh, sks_sh,
               sem_l0, sem_l1, sem_g0, sem_g1, sem_s0, sem_s1):
    c = lax.axis_index("c")
    s = lax.axis_index("s")
    g = c * NS + s
    zero16 = jnp.zeros((16,), jnp.float32)
    bufs = (
        (rows0, exb0, siv0, sjv0, qidx0, ridx0, didx0,
         sem_l0, sem_g0, sem_s0),
        (rows1, exb1, siv1, sjv1, qidx1, ridx1, didx1,
         sem_l1, sem_g1, sem_s1))

    pltpu.sync_copy(sh_ref.at[0, pl.ds(0, 16)], shv_v)

    @pl.when(s < R)
    def _():
        pltpu.sync_copy(sq_ref.at[pl.ds(s * N, N)], den_v)
        pltpu.sync_copy(den_v, sqs_sh.at[pl.ds(s * N, N)])

    @pl.when((s >= R) & (s < 2 * R))
    def _():
        pltpu.sync_copy(sk_ref.at[pl.ds((s - R) * N, N)], den_v)
        pltpu.sync_copy(den_v, sks_sh.at[pl.ds((s - R) * N, N)])

    def _zr(i, carry):
        for j in range(D // 16):
            rows0[i, pl.ds(j * 16, 16)] = zero16
        return carry
    lax.fori_loop(0, CH, _zr, 0)
    for j in range(-(-NRC // NS)):
        cid = s + j * NS

        @pl.when(cid < NRC)
        def _():
            pltpu.sync_copy(rows0, acc_sh.at[pl.ds(cid * CH, CH)])

    @pl.when(s == 0)
    def _():
        def _zd(i, carry):
            den_v[pl.ds(i * 16, 16)] = zero16
            return carry
        lax.fori_loop(0, N // 16, _zd, 0)
        pltpu.sync_copy(den_v, den_sh)

    plsc.subcore_barrier()

    shv = shv_v[...]

    def _wait_scatters(b):
        rows_v, exb, _, _, _, _, didx, _, _, sem_s = bufs[b]
        pltpu.make_async_copy(rows_v, acc_sh.at[didx.at[0]], sem_s).wait()
        pltpu.make_async_copy(exb, den_sh.at[didx.at[0]], sem_s).wait()

    def _stage_a(i, b, guard):
        rows_v, exb, siv, sjv, qidx, ridx, didx, sem_l, sem_g, _ = bufs[b]
        if guard:
            @pl.when(i >= 2)
            def _():
                _wait_scatters(b)
        cb = i * CH
        for gi in range(CH // 16):
            o = cb + gi * 16
            sv = src_v[pl.ds(o, 16)]
            dv = dst_v[pl.ds(o, 16)]
            tv = ty_v[pl.ds(o, 16)]
            tn = tv * N
            qidx[0, pl.ds(gi * 16, 16)] = tn + dv
            ridx[0, pl.ds(gi * 16, 16)] = tn + sv
            didx[0, pl.ds(gi * 16, 16)] = dv
        pltpu.async_copy(sqs_sh.at[qidx.at[0]], siv, sem_l)
        pltpu.async_copy(sks_sh.at[ridx.at[0]], sjv, sem_l)
        pltpu.async_copy(xt_ref.at[ridx.at[0]], rows_v, sem_g)

    def _stage_b(i, b):
        rows_v, exb, siv, sjv, qidx, ridx, didx, sem_l, sem_g, sem_s = bufs[b]
        pltpu.make_async_copy(sqs_sh.at[qidx.at[0]], siv, sem_l).wait()
        pltpu.make_async_copy(sks_sh.at[ridx.at[0]], sjv, sem_l).wait()

        for gi in range(CH // 16):
            si = siv[pl.ds(gi * 16, 16)]
            sj = sjv[pl.ds(gi * 16, 16)]
            al = si + sj
            al = jnp.where(al >= 0, al, 0.2 * al)
            exb[pl.ds(gi * 16, 16)] = jnp.exp(al - shv)

        pltpu.make_async_copy(xt_ref.at[ridx.at[0]], rows_v, sem_g).wait()

        def _scale(g2, carry2):
            ex16 = exb[pl.ds(g2 * 16, 16)]
            for j in range(16):
                m16 = lax.gather(
                    ex16, jnp.full((16, 1), j, jnp.int32),
                    _GDN, (1,),
                    mode=lax.GatherScatterMode.PROMISE_IN_BOUNDS)
                ci = g2 * 16 + j
                for jj in range(D // 16):
                    rows_v[ci, pl.ds(jj * 16, 16)] = (
                        rows_v[ci, pl.ds(jj * 16, 16)] * m16)
            return carry2
        lax.fori_loop(0, CH // 16, _scale, 0)

        pltpu.async_copy(rows_v, acc_sh.at[didx.at[0]], sem_s, add=True)
        pltpu.async_copy(exb, den_sh.at[didx.at[0]], sem_s, add=True)

    def _sup(sup, carry0):
        b0 = g * EPT + sup * SUP
        pltpu.sync_copy(esrc_ref.at[pl.ds(b0, SUP)], src_v)
        pltpu.sync_copy(edst_ref.at[pl.ds(b0, SUP)], dst_v)
        pltpu.sync_copy(et_ref.at[pl.ds(b0, SUP)], ty_v)

        _stage_a(0, 0, False)

        def _pair(p, c2):
            _stage_a(2 * p + 1, 1, True)
            _stage_b(2 * p, 0)
            _stage_a(2 * p + 2, 0, True)
            _stage_b(2 * p + 1, 1)
            return c2
        lax.fori_loop(0, (NCHUNK - 1) // 2, _pair, 0)
        _stage_b(NCHUNK - 1, 0)
        _wait_scatters(1)
        _wait_scatters(0)
        return carry0
    lax.fori_loop(0, NSUP, _sup, 0)

    plsc.subcore_barrier()

    for j in range(-(-NRC // NS)):
        cid = s + j * NS

        @pl.when(cid < NRC)
        def _():
            pltpu.sync_copy(acc_sh.at[pl.ds(cid * CH, CH)], rows0)
            pltpu.sync_copy(rows0, acc_ref.at[c, pl.ds(cid * CH, CH)])

    @pl.when(s == 0)
    def _():
        pltpu.sync_copy(den_sh, den_v)
        pltpu.sync_copy(den_v, den_ref.at[pl.ds(c * N, N)])


def _edge_pass(esrc, edst, et, sqf, skf, shift, xt2):
    mesh = plsc.VectorSubcoreMesh(core_axis_name="c", subcore_axis_name="s")
    f = pl.kernel(
        _edge_body,
        mesh=mesh,
        out_type=[
            jax.ShapeDtypeStruct((NC, N, D), jnp.float32),
            jax.ShapeDtypeStruct((NC * N,), jnp.float32),
        ],
        scratch_types=[
            pltpu.VMEM((16,), jnp.float32),
            pltpu.VMEM((SUP,), jnp.int32),
            pltpu.VMEM((SUP,), jnp.int32),
            pltpu.VMEM((SUP,), jnp.int32),
            pltpu.VMEM((CH, D), jnp.float32),
            pltpu.VMEM((CH, D), jnp.float32),
            pltpu.VMEM((N,), jnp.float32),
            pltpu.VMEM((CH,), jnp.float32),
            pltpu.VMEM((CH,), jnp.float32),
            pltpu.VMEM((CH,), jnp.float32),
            pltpu.VMEM((CH,), jnp.float32),
            pltpu.VMEM((CH,), jnp.float32),
            pltpu.VMEM((CH,), jnp.float32),
            pltpu.VMEM((1, CH), jnp.int32),
            pltpu.VMEM((1, CH), jnp.int32),
            pltpu.VMEM((1, CH), jnp.int32),
            pltpu.VMEM((1, CH), jnp.int32),
            pltpu.VMEM((1, CH), jnp.int32),
            pltpu.VMEM((1, CH), jnp.int32),
            pltpu.VMEM_SHARED((N, D), jnp.float32),
            pltpu.VMEM_SHARED((N,), jnp.float32),
            pltpu.VMEM_SHARED((R * N,), jnp.float32),
            pltpu.VMEM_SHARED((R * N,), jnp.float32),
            pltpu.SemaphoreType.DMA,
            pltpu.SemaphoreType.DMA,
            pltpu.SemaphoreType.DMA,
            pltpu.SemaphoreType.DMA,
            pltpu.SemaphoreType.DMA,
            pltpu.SemaphoreType.DMA,
        ],
    )
    return f(esrc, edst, et, sqf, skf, shift, xt2)


def _rgat_layer(h, ei, et, w, q, k, b, do_relu):
    xt, sq3, sk3, shift = _prep(h, w, q, k)
    xt2 = xt.reshape(R * N, D)
    acc, den = _edge_pass(ei[0], ei[1], et, sq3.reshape(R * N),
                          sk3.reshape(R * N), shift, xt2)
    return _finalize(acc, den.reshape(NC, N), b, do_relu)


def kernel(z, edge_index, edge_type, W_lin, b_lin, w1, q1, k1, b1,
           w2, q2, k2, b2):
    h = _linear(z, W_lin, b_lin)
    h = _rgat_layer(h, edge_index, edge_type, w1, q1, k1, b1, True)
    return _rgat_layer(h, edge_index, edge_type, w2, q2, k2, b2, False)
